# Initial kernel scaffold; baseline (speedup 1.0000x reference)
import jax
import jax.numpy as jnp
from jax.experimental import pallas as pl


def kernel(cat_indices, embed_table, lr_weight, lr_bias):
    raise NotImplementedError("write your pallas kernel here")


# speedup vs baseline: 7.5715x; 7.5715x over previous
import functools

import jax
import jax.numpy as jnp
from jax import lax
from jax.experimental import pallas as pl
from jax.experimental.pallas import tpu as pltpu
from jax.experimental.pallas import tpu_sc as plsc

_NC = 2
_NS = 16
_NW = _NC * _NS
_L = 16

_CHUNK = 64


@functools.partial(jax.jit, static_argnames=("B", "F", "V", "D"))
def _fm_sc(cat_flat, emb, lr_flat, bias, *, B, F, V, D):
    rows_per_w = B // _NW
    n_chunks = rows_per_w // _CHUNK
    idx_len = _CHUNK * F

    mesh = plsc.VectorSubcoreMesh(core_axis_name="c", subcore_axis_name="s")

    @functools.partial(
        pl.kernel,
        out_type=jax.ShapeDtypeStruct((B,), jnp.float32),
        mesh=mesh,
        compiler_params=pltpu.CompilerParams(needs_layout_passes=False,
                                             use_tc_tiling_on_sc=False),
        scratch_types=[
            pltpu.VMEM((idx_len,), jnp.int32),
            pltpu.VMEM((F, _CHUNK), jnp.int32),
            pltpu.VMEM((F * _CHUNK, D), jnp.float32),
            pltpu.VMEM((F, _CHUNK), jnp.float32),
            pltpu.VMEM((B // _NW,), jnp.float32),
            pltpu.SemaphoreType.DMA,
        ],
    )
    def fm_kernel(cat_hbm, emb_hbm, lr_hbm, out_hbm,
                  idxraw_v, idx_v, ebuf, lbuf, out_v, sem):
        wid = lax.axis_index("s") * _NC + lax.axis_index("c")
        w_base = wid * (rows_per_w * F)

        zeros16 = jnp.zeros((_L,), jnp.float32)

        jlane = lax.iota(jnp.int32, _L)
        jF = jlane * F

        def chunk_body(c, carry):
            src_off = pl.multiple_of(w_base + c * idx_len, idx_len)
            pltpu.sync_copy(cat_hbm.at[pl.ds(src_off, idx_len)], idxraw_v)

            for f in range(F):
                for g in range(_CHUNK // _L):
                    addr = jF + (g * _L * F + f)
                    vals = plsc.load_gather(idxraw_v, [addr])
                    idx_v[f, pl.ds(g * _L, _L)] = vals + (f * V)

            copies = []
            for f in range(F):
                copies.append(pltpu.async_copy(
                    emb_hbm.at[idx_v.at[f]],
                    ebuf.at[pl.ds(f * _CHUNK, _CHUNK)], sem))
                copies.append(pltpu.async_copy(
                    lr_hbm.at[idx_v.at[f]], lbuf.at[f], sem))
            for cp in copies:
                cp.wait()

            for g in range(_CHUNK // _L):
                jrow = jlane + (g * _L)
                rowv = [jrow + f * _CHUNK for f in range(F)]

                def d_body(d, acc):
                    ss, q = acc
                    dcol = jnp.broadcast_to(d, (_L,))
                    t = zeros16
                    for f in range(F):
                        e = plsc.load_gather(ebuf, [rowv[f], dcol])
                        t = t + e
                        q = q + e * e
                    return ss + t * t, q

                ss, q = lax.fori_loop(0, D, d_body, (zeros16, zeros16))

                fo = zeros16
                for f in range(F):
                    fo = fo + lbuf[f, pl.ds(g * _L, _L)]

                res = 0.5 * (ss - q) + fo
                dst = pl.multiple_of(c * _CHUNK + g * _L, _L)
                out_v[pl.ds(dst, _L)] = res
            return carry

        lax.fori_loop(0, n_chunks, chunk_body, 0)

        out_off = pl.multiple_of(wid * rows_per_w, rows_per_w)
        pltpu.sync_copy(out_v, out_hbm.at[pl.ds(out_off, rows_per_w)])

    return fm_kernel(cat_flat, emb, lr_flat) + bias


def kernel(cat_indices, embed_table, lr_weight, lr_bias):
    B, F = cat_indices.shape
    D = embed_table.shape[1]
    V = embed_table.shape[0] // F
    assert B % (_NW * _CHUNK) == 0 and D == _L

    cat_flat = cat_indices.astype(jnp.int32).reshape(B * F)
    lr_flat = lr_weight.reshape(-1)
    out = _fm_sc(cat_flat, embed_table, lr_flat, lr_bias, B=B, F=F, V=V, D=D)
    return out[:, None]
